# Initial kernel scaffold; baseline (speedup 1.0000x reference)
import jax
import jax.numpy as jnp
from jax.experimental import pallas as pl


def kernel(input_data, edge_index, W1, b1, W2, b2):
    raise NotImplementedError("write your pallas kernel here")


# speedup vs baseline: 4.5764x; 4.5764x over previous
import functools

import jax
import jax.numpy as jnp
from jax import lax
from jax.experimental import pallas as pl
from jax.experimental.pallas import tpu as pltpu
from jax.experimental.pallas import tpu_sc as plsc

N_NODES = 50000
N_EDGES = 800000
LANE = 128
ROWS = N_EDGES // LANE
NC, NS = 2, 16
NPAD = 50176
HALF = NPAD // 2
SPAD = HALF + 128
DUMMY = HALF + 64
PT = HALF // NS
ZT = SPAD // NS
DEG_W = 51200
DEG_T = DEG_W // NS
ROWS_A = ROWS // NC
TRIPS_A = (ROWS_A + NS - 1) // NS
TRIPS_S = (ROWS + NS - 1) // NS


def _mesh():
    return plsc.VectorSubcoreMesh(
        core_axis_name="c", subcore_axis_name="s", num_cores=NC, num_subcores=NS
    )


def _deg_body(e_hbm, out_hbm, idxb, ones_v, zbuf, deg_sp):
    c = lax.axis_index("c")
    s = lax.axis_index("s")

    def zstep(i, _):
        zbuf[pl.ds(i * 16, 16)] = jnp.zeros((16,), jnp.float32)
        return 0

    lax.fori_loop(0, DEG_T // 16, zstep, 0)
    pltpu.sync_copy(zbuf, deg_sp.at[pl.ds(s * DEG_T, DEG_T)])
    for k in range(LANE // 16):
        ones_v[pl.ds(k * 16, 16)] = jnp.ones((16,), jnp.float32)
    plsc.subcore_barrier()

    def step(i, _):
        r = i * NS + s

        @pl.when(r < ROWS_A)
        def _():
            row = c * ROWS_A + r
            pltpu.sync_copy(e_hbm.at[0, row, 0], idxb.at[0])
            pltpu.sync_copy(ones_v, deg_sp.at[idxb.at[0]], add=True)

        return 0

    lax.fori_loop(0, TRIPS_A, step, 0)
    plsc.subcore_barrier()
    pltpu.sync_copy(
        deg_sp.at[pl.ds(s * DEG_T, DEG_T)],
        out_hbm.at[pl.ds(c * DEG_W + s * DEG_T, DEG_T)],
    )


_deg_call = pl.kernel(
    _deg_body,
    out_type=jax.ShapeDtypeStruct((NC * DEG_W,), jnp.float32),
    mesh=_mesh(),
    compiler_params=pltpu.CompilerParams(use_tc_tiling_on_sc=False),
    scratch_types=[
        pltpu.VMEM((1, LANE), jnp.int32),
        pltpu.VMEM((LANE,), jnp.float32),
        pltpu.VMEM((DEG_T,), jnp.float32),
        pltpu.VMEM_SHARED((DEG_W,), jnp.float32),
    ],
)


def _make_segsum(D):
    def body(e_hbm, tab_hbm, zeros_hbm, out_hbm, sidx, didx, rows_v, sem, sp):
        c = lax.axis_index("c")
        s = lax.axis_index("s")
        lo = c * HALF

        pltpu.sync_copy(zeros_hbm.at[pl.ds(s * ZT, ZT)], sp.at[pl.ds(s * ZT, ZT)])
        plsc.subcore_barrier()

        def step(i, _):
            r = i * NS + s

            @pl.when(r < ROWS)
            def _():
                pltpu.sync_copy(e_hbm.at[0, r, 0], sidx.at[0])
                pltpu.sync_copy(e_hbm.at[1, r, 0], didx.at[0])
                for k in range(LANE // 16):
                    v = didx[0, pl.ds(k * 16, 16)]
                    ok = (v >= lo) & (v < lo + HALF)
                    didx[0, pl.ds(k * 16, 16)] = jnp.where(ok, v - lo, DUMMY)
                pltpu.async_copy(tab_hbm.at[sidx.at[0]], rows_v, sem).wait()
                pltpu.sync_copy(rows_v, sp.at[didx.at[0]], add=True)

            return 0

        lax.fori_loop(0, TRIPS_S, step, 0)
        plsc.subcore_barrier()
        pltpu.sync_copy(
            sp.at[pl.ds(s * PT, PT)], out_hbm.at[pl.ds(c * HALF + s * PT, PT)]
        )

    return pl.kernel(
        body,
        out_type=jax.ShapeDtypeStruct((NPAD, D), jnp.float32),
        mesh=_mesh(),
        compiler_params=pltpu.CompilerParams(use_tc_tiling_on_sc=False),
        scratch_types=[
            pltpu.VMEM((1, LANE), jnp.int32),
            pltpu.VMEM((1, LANE), jnp.int32),
            pltpu.VMEM((LANE, D), jnp.float32),
            pltpu.SemaphoreType.DMA,
            pltpu.VMEM_SHARED((SPAD, D), jnp.float32),
        ],
    )


_segsum8 = _make_segsum(8)
_segsum64 = _make_segsum(64)


_RB = 6272


def _b_body(x_ref, dT_ref, o_ref):
    d = dT_ref[:, 0:1] + dT_ref[:, 1:2]
    no = lax.rsqrt(jnp.maximum(d, 1.0))
    xn = x_ref[...] * no
    zz = jnp.zeros((_RB, 1), jnp.float32)
    oo = jnp.ones((_RB, 1), jnp.float32)
    o_ref[...] = jnp.concatenate([xn, zz, oo], axis=1)


def _pass_b(xpad, degT):
    return pl.pallas_call(
        _b_body,
        grid=(NPAD // _RB,),
        in_specs=[
            pl.BlockSpec((_RB, 6), lambda i: (i, 0)),
            pl.BlockSpec((_RB, 2), lambda i: (i, 0)),
        ],
        out_specs=pl.BlockSpec((_RB, 8), lambda i: (i, 0)),
        out_shape=jax.ShapeDtypeStruct((NPAD, 8), jnp.float32),
    )(xpad, degT)


def _d_body(a_ref, dT_ref, w_ref, b_ref, o_ref):
    a = a_ref[...]
    ni = lax.rsqrt(jnp.maximum(a[:, 7:8], 1.0))
    dout = dT_ref[:, 0:1] + dT_ref[:, 1:2]
    no = lax.rsqrt(jnp.maximum(dout, 1.0))
    h = jnp.dot(a, w_ref[...], preferred_element_type=jnp.float32)
    o_ref[...] = jnp.maximum(h * ni + b_ref[...], 0.0) * no


def _pass_d(agg1, degT, W1p, b1r):
    return pl.pallas_call(
        _d_body,
        grid=(NPAD // _RB,),
        in_specs=[
            pl.BlockSpec((_RB, 8), lambda i: (i, 0)),
            pl.BlockSpec((_RB, 2), lambda i: (i, 0)),
            pl.BlockSpec((8, 64), lambda i: (0, 0)),
            pl.BlockSpec((1, 64), lambda i: (0, 0)),
        ],
        out_specs=pl.BlockSpec((_RB, 64), lambda i: (i, 0)),
        out_shape=jax.ShapeDtypeStruct((NPAD, 64), jnp.float32),
    )(agg1, degT, W1p, b1r)


_RF = 2000


def _f_body(a2_ref, a1_ref, w_ref, b_ref, o_ref):
    ni = lax.rsqrt(jnp.maximum(a1_ref[:, 7:8], 1.0))
    h = jnp.dot(a2_ref[...] * ni, w_ref[...], preferred_element_type=jnp.float32)
    o_ref[...] = jnp.maximum(h + b_ref[...], 0.0)


def _pass_f(agg2, agg1, W2, b2r):
    return pl.pallas_call(
        _f_body,
        grid=(N_NODES // _RF,),
        in_specs=[
            pl.BlockSpec((_RF, 64), lambda i: (i, 0)),
            pl.BlockSpec((_RF, 8), lambda i: (i, 0)),
            pl.BlockSpec((64, 50), lambda i: (0, 0)),
            pl.BlockSpec((1, 50), lambda i: (0, 0)),
        ],
        out_specs=pl.BlockSpec((_RF, 50), lambda i: (i, 0)),
        out_shape=jax.ShapeDtypeStruct((N_NODES, 50), jnp.float32),
    )(agg2, agg1, W2, b2r)


def kernel(input_data, edge_index, W1, b1, W2, b2):
    e3 = edge_index.reshape(2, ROWS, 1, LANE)
    deg2 = _deg_call(e3).reshape(NC, DEG_W)
    degT = jnp.transpose(deg2)[:NPAD]
    xpad = jnp.pad(input_data, ((0, NPAD - N_NODES), (0, 0)))
    xext = _pass_b(xpad, degT)
    z8 = jnp.zeros((SPAD, 8), jnp.float32)
    agg1 = _segsum8(e3, xext, z8)
    W1p = jnp.pad(W1, ((0, 2), (0, 0)))
    h1n = _pass_d(agg1, degT, W1p, b1.reshape(1, 64))
    z64 = jnp.zeros((SPAD, 64), jnp.float32)
    agg2 = _segsum64(e3, h1n, z64)
    return _pass_f(agg2, agg1, W2, b2.reshape(1, 50))


# speedup vs baseline: 6.1384x; 1.3413x over previous
import functools

import jax
import jax.numpy as jnp
from jax import lax
from jax.experimental import pallas as pl
from jax.experimental.pallas import tpu as pltpu
from jax.experimental.pallas import tpu_sc as plsc

N_NODES = 50000
N_EDGES = 800000
LANE = 128
ROWS = N_EDGES // LANE
ROWS_P = 6272
PAD_SRC = 50000
PAD_DST = 50100
NC, NS = 2, 16
NPAD = 50176
HALF = NPAD // 2
SPAD = HALF + 128
DUMMY = HALF + 64
PT = HALF // NS
ZT = SPAD // NS
DEG_W = 51200
DEG_T = DEG_W // NS
ROWS_A = ROWS_P // NC
TRIPS_A = ROWS_A // NS
TRIPS_S = ROWS_P // NS


def _mesh():
    return plsc.VectorSubcoreMesh(
        core_axis_name="c", subcore_axis_name="s", num_cores=NC, num_subcores=NS
    )


def _deg_body(e_hbm, out_hbm, idxb, ones_v, zbuf, ssem, deg_sp):
    c = lax.axis_index("c")
    s = lax.axis_index("s")

    def zstep(i, _):
        zbuf[pl.ds(i * 16, 16)] = jnp.zeros((16,), jnp.float32)
        return 0

    lax.fori_loop(0, DEG_T // 16, zstep, 0)
    pltpu.sync_copy(zbuf, deg_sp.at[pl.ds(s * DEG_T, DEG_T)])
    for k in range(LANE // 16):
        ones_v[pl.ds(k * 16, 16)] = jnp.ones((16,), jnp.float32)
    plsc.subcore_barrier()

    def step(g, _):
        for b in range(2):
            @pl.when(g >= 1)
            def _():
                pltpu.make_async_copy(ones_v, deg_sp.at[idxb.at[b]], ssem[b]).wait()

            row = c * ROWS_A + (g * 2 + b) * NS + s
            pltpu.sync_copy(e_hbm.at[0, row, 0], idxb.at[b])
            pltpu.async_copy(ones_v, deg_sp.at[idxb.at[b]], ssem[b], add=True)
        return 0

    lax.fori_loop(0, TRIPS_A // 2, step, 0)
    for b in range(2):
        pltpu.make_async_copy(ones_v, deg_sp.at[idxb.at[b]], ssem[b]).wait()
    plsc.subcore_barrier()
    pltpu.sync_copy(
        deg_sp.at[pl.ds(s * DEG_T, DEG_T)],
        out_hbm.at[pl.ds(c * DEG_W + s * DEG_T, DEG_T)],
    )


_deg_call = pl.kernel(
    _deg_body,
    out_type=jax.ShapeDtypeStruct((NC * DEG_W,), jnp.float32),
    mesh=_mesh(),
    compiler_params=pltpu.CompilerParams(use_tc_tiling_on_sc=False),
    scratch_types=[
        pltpu.VMEM((2, LANE), jnp.int32),
        pltpu.VMEM((LANE,), jnp.float32),
        pltpu.VMEM((DEG_T,), jnp.float32),
        [pltpu.SemaphoreType.DMA, pltpu.SemaphoreType.DMA],
        pltpu.VMEM_SHARED((DEG_W,), jnp.float32),
    ],
)


def _make_segsum(D):
    def body(e_hbm, tab_hbm, zeros_hbm, out_hbm, sidx, didx, rows_v, gsem, ssem, sp):
        c = lax.axis_index("c")
        s = lax.axis_index("s")
        lo = c * HALF

        pltpu.sync_copy(zeros_hbm.at[pl.ds(s * ZT, ZT)], sp.at[pl.ds(s * ZT, ZT)])
        plsc.subcore_barrier()

        def step(g, _):
            for b in range(2):
                bp = 1 - b

                @pl.when(g >= 1)
                def _():
                    pltpu.make_async_copy(
                        rows_v.at[b], sp.at[didx.at[b]], ssem[b]
                    ).wait()

                r = (g * 2 + b) * NS + s
                pltpu.sync_copy(e_hbm.at[0, r, 0], sidx.at[b])
                pltpu.sync_copy(e_hbm.at[1, r, 0], didx.at[b])
                for k in range(LANE // 16):
                    v = didx[b, pl.ds(k * 16, 16)]
                    ok = (v >= lo) & (v < lo + HALF)
                    didx[b, pl.ds(k * 16, 16)] = jnp.where(ok, v - lo, DUMMY)
                pltpu.async_copy(tab_hbm.at[sidx.at[b]], rows_v.at[b], gsem[b])

                def retire():
                    pltpu.make_async_copy(
                        tab_hbm.at[sidx.at[bp]], rows_v.at[bp], gsem[bp]
                    ).wait()
                    pltpu.async_copy(
                        rows_v.at[bp], sp.at[didx.at[bp]], ssem[bp], add=True
                    )

                if b == 1:
                    retire()
                else:
                    pl.when(g >= 1)(retire)
            return 0

        lax.fori_loop(0, TRIPS_S // 2, step, 0)
        pltpu.make_async_copy(tab_hbm.at[sidx.at[1]], rows_v.at[1], gsem[1]).wait()
        pltpu.async_copy(rows_v.at[1], sp.at[didx.at[1]], ssem[1], add=True)
        pltpu.make_async_copy(rows_v.at[0], sp.at[didx.at[0]], ssem[0]).wait()
        pltpu.make_async_copy(rows_v.at[1], sp.at[didx.at[1]], ssem[1]).wait()
        plsc.subcore_barrier()
        pltpu.sync_copy(
            sp.at[pl.ds(s * PT, PT)], out_hbm.at[pl.ds(c * HALF + s * PT, PT)]
        )

    return pl.kernel(
        body,
        out_type=jax.ShapeDtypeStruct((NPAD, D), jnp.float32),
        mesh=_mesh(),
        compiler_params=pltpu.CompilerParams(use_tc_tiling_on_sc=False),
        scratch_types=[
            pltpu.VMEM((2, LANE), jnp.int32),
            pltpu.VMEM((2, LANE), jnp.int32),
            pltpu.VMEM((2, LANE, D), jnp.float32),
            [pltpu.SemaphoreType.DMA, pltpu.SemaphoreType.DMA],
            [pltpu.SemaphoreType.DMA, pltpu.SemaphoreType.DMA],
            pltpu.VMEM_SHARED((SPAD, D), jnp.float32),
        ],
    )


_segsum8 = _make_segsum(8)
_segsum64 = _make_segsum(64)


_RB = 6272


def _b_body(x_ref, dT_ref, o_ref):
    d = dT_ref[:, 0:1] + dT_ref[:, 1:2]
    no = lax.rsqrt(jnp.maximum(d, 1.0))
    xn = x_ref[...] * no
    zz = jnp.zeros((_RB, 1), jnp.float32)
    oo = jnp.ones((_RB, 1), jnp.float32)
    o_ref[...] = jnp.concatenate([xn, zz, oo], axis=1)


def _pass_b(xpad, degT):
    return pl.pallas_call(
        _b_body,
        grid=(NPAD // _RB,),
        in_specs=[
            pl.BlockSpec((_RB, 6), lambda i: (i, 0)),
            pl.BlockSpec((_RB, 2), lambda i: (i, 0)),
        ],
        out_specs=pl.BlockSpec((_RB, 8), lambda i: (i, 0)),
        out_shape=jax.ShapeDtypeStruct((NPAD, 8), jnp.float32),
    )(xpad, degT)


def _d_body(a_ref, dT_ref, w_ref, b_ref, o_ref):
    a = a_ref[...]
    ni = lax.rsqrt(jnp.maximum(a[:, 7:8], 1.0))
    dout = dT_ref[:, 0:1] + dT_ref[:, 1:2]
    no = lax.rsqrt(jnp.maximum(dout, 1.0))
    h = jnp.dot(a, w_ref[...], preferred_element_type=jnp.float32)
    o_ref[...] = jnp.maximum(h * ni + b_ref[...], 0.0) * no


def _pass_d(agg1, degT, W1p, b1r):
    return pl.pallas_call(
        _d_body,
        grid=(NPAD // _RB,),
        in_specs=[
            pl.BlockSpec((_RB, 8), lambda i: (i, 0)),
            pl.BlockSpec((_RB, 2), lambda i: (i, 0)),
            pl.BlockSpec((8, 64), lambda i: (0, 0)),
            pl.BlockSpec((1, 64), lambda i: (0, 0)),
        ],
        out_specs=pl.BlockSpec((_RB, 64), lambda i: (i, 0)),
        out_shape=jax.ShapeDtypeStruct((NPAD, 64), jnp.float32),
    )(agg1, degT, W1p, b1r)


_RF = 2000


def _f_body(a2_ref, a1_ref, w_ref, b_ref, o_ref):
    ni = lax.rsqrt(jnp.maximum(a1_ref[:, 7:8], 1.0))
    h = jnp.dot(a2_ref[...] * ni, w_ref[...], preferred_element_type=jnp.float32)
    o_ref[...] = jnp.maximum(h + b_ref[...], 0.0)


def _pass_f(agg2, agg1, W2, b2r):
    return pl.pallas_call(
        _f_body,
        grid=(N_NODES // _RF,),
        in_specs=[
            pl.BlockSpec((_RF, 64), lambda i: (i, 0)),
            pl.BlockSpec((_RF, 8), lambda i: (i, 0)),
            pl.BlockSpec((64, 50), lambda i: (0, 0)),
            pl.BlockSpec((1, 50), lambda i: (0, 0)),
        ],
        out_specs=pl.BlockSpec((_RF, 50), lambda i: (i, 0)),
        out_shape=jax.ShapeDtypeStruct((N_NODES, 50), jnp.float32),
    )(agg2, agg1, W2, b2r)


def kernel(input_data, edge_index, W1, b1, W2, b2):
    npad_e = ROWS_P * LANE - N_EDGES
    fill = jnp.array([[PAD_SRC], [PAD_DST]], jnp.int32)
    epad = jnp.concatenate(
        [edge_index, jnp.broadcast_to(fill, (2, npad_e))], axis=1
    )
    e3 = epad.reshape(2, ROWS_P, 1, LANE)
    deg2 = _deg_call(e3).reshape(NC, DEG_W)
    degT = jnp.transpose(deg2)[:NPAD]
    xpad = jnp.pad(input_data, ((0, NPAD - N_NODES), (0, 0)))
    xext = _pass_b(xpad, degT)
    z8 = jnp.zeros((SPAD, 8), jnp.float32)
    agg1 = _segsum8(e3, xext, z8)
    W1p = jnp.pad(W1, ((0, 2), (0, 0)))
    h1n = _pass_d(agg1, degT, W1p, b1.reshape(1, 64))
    z64 = jnp.zeros((SPAD, 64), jnp.float32)
    agg2 = _segsum64(e3, h1n, z64)
    return _pass_f(agg2, agg1, W2, b2.reshape(1, 50))


# speedup vs baseline: 10.7945x; 1.7585x over previous
import jax
import jax.numpy as jnp
from jax import lax
from jax.experimental import pallas as pl
from jax.experimental.pallas import tpu as pltpu
from jax.experimental.pallas import tpu_sc as plsc

N_NODES = 50000
N_EDGES = 800000
LANE = 128
NC, NS = 2, 16
ROWS_P = 6400
PAD_SRC = 50000
PAD_DST = 50100
NPAD = 50176
ZT = NPAD // NS
DEG_W = 51200
DEG_T = DEG_W // NS

IBLK = 20
NSLOT = 4

T_E = ROWS_P // NS
NBLK_E = T_E // IBLK
T_H = ROWS_P // (NC * NS)
NBLK_H = T_H // IBLK


def _mesh():
    return plsc.VectorSubcoreMesh(
        core_axis_name="c", subcore_axis_name="s", num_cores=NC, num_subcores=NS
    )


def _sc_params():
    return pltpu.CompilerParams(use_tc_tiling_on_sc=False)


def _pipeline(e_hbm, base, nblk, ibuf, isem, prep, fire, retire, drain):
    def iload(k_dyn, kslot):
        pltpu.async_copy(
            e_hbm.at[pl.ds(base + k_dyn * IBLK, IBLK)], ibuf.at[kslot], isem[kslot]
        )

    def iwait(k_dyn, kslot):
        pltpu.make_async_copy(
            e_hbm.at[pl.ds(base + k_dyn * IBLK, IBLK)], ibuf.at[kslot], isem[kslot]
        ).wait()

    iload(0, 0)

    def blk(g, _):
        for bb in range(2):
            k = g * 2 + bb
            iwait(k, bb)
            for j2 in range(IBLK):
                p = j2 % NSLOT
                if j2 >= NSLOT:
                    drain(bb, j2 - NSLOT, p)
                else:
                    def _dr(bb=bb, j2=j2, p=p):
                        drain(1 - bb, IBLK - NSLOT + j2, p)

                    pl.when(k >= 1)(_dr)
                prep(bb, j2)
                fire(bb, j2, p)
                q = (j2 - 2) % NSLOT
                if j2 >= 2:
                    retire(bb, j2 - 2, q)
                else:
                    def _rt(bb=bb, j2=j2, q=q):
                        retire(1 - bb, IBLK - 2 + j2, q)

                    pl.when(k >= 1)(_rt)
                if j2 == NSLOT - 1:
                    def _ld(k=k, bb=bb):
                        iload(k + 1, 1 - bb)

                    pl.when(k < nblk - 1)(_ld)
        return 0

    lax.fori_loop(0, nblk // 2, blk, 0)
    retire(1, IBLK - 2, (IBLK - 2) % NSLOT)
    retire(1, IBLK - 1, (IBLK - 1) % NSLOT)
    for dj in range(NSLOT):
        j2 = IBLK - NSLOT + dj
        drain(1, j2, j2 % NSLOT)


def _deg_body(e_hbm, out_hbm, ibuf, ones_v, zbuf, isem, ssem, deg_sp):
    c = lax.axis_index("c")
    s = lax.axis_index("s")

    def zstep(i, _):
        zbuf[pl.ds(i * 16, 16)] = jnp.zeros((16,), jnp.float32)
        return 0

    lax.fori_loop(0, DEG_T // 16, zstep, 0)
    pltpu.sync_copy(zbuf, deg_sp.at[pl.ds(s * DEG_T, DEG_T)])
    for m in range(LANE // 16):
        ones_v[pl.ds(m * 16, 16)] = jnp.ones((16,), jnp.float32)
    plsc.subcore_barrier()

    base = (c * NS + s) * T_H

    def prep(kslot, j2):
        pass

    def fire(kslot, j2, p):
        pltpu.async_copy(ones_v, deg_sp.at[ibuf.at[kslot, j2, 0]], ssem[p], add=True)

    def retire(kslot, j2, p):
        pass

    def drain(kslot, j2, p):
        pltpu.make_async_copy(ones_v, deg_sp.at[ibuf.at[kslot, j2, 0]], ssem[p]).wait()

    _pipeline(e_hbm, base, NBLK_H, ibuf, isem, prep, fire, retire, drain)
    plsc.subcore_barrier()
    pltpu.sync_copy(
        deg_sp.at[pl.ds(s * DEG_T, DEG_T)],
        out_hbm.at[pl.ds(c * DEG_W + s * DEG_T, DEG_T)],
    )


_deg_call = pl.kernel(
    _deg_body,
    out_type=jax.ShapeDtypeStruct((NC * DEG_W,), jnp.float32),
    mesh=_mesh(),
    compiler_params=_sc_params(),
    scratch_types=[
        pltpu.VMEM((2, IBLK, 2, LANE), jnp.int32),
        pltpu.VMEM((LANE,), jnp.float32),
        pltpu.VMEM((DEG_T,), jnp.float32),
        [pltpu.SemaphoreType.DMA] * 2,
        [pltpu.SemaphoreType.DMA] * NSLOT,
        pltpu.VMEM_SHARED((DEG_W,), jnp.float32),
    ],
)


def _segsum8_body(e_hbm, tab_hbm, zeros_hbm, out_hbm, ibuf, rows_v, isem, gsem, ssem, sp):
    c = lax.axis_index("c")
    s = lax.axis_index("s")

    pltpu.sync_copy(zeros_hbm.at[pl.ds(s * ZT, ZT)], sp.at[pl.ds(s * ZT, ZT)])
    plsc.subcore_barrier()

    base = (c * NS + s) * T_H

    def prep(kslot, j2):
        pass

    def fire(kslot, j2, p):
        pltpu.async_copy(tab_hbm.at[ibuf.at[kslot, j2, 0]], rows_v.at[p], gsem[p])

    def retire(kslot, j2, p):
        pltpu.make_async_copy(
            tab_hbm.at[ibuf.at[kslot, j2, 0]], rows_v.at[p], gsem[p]
        ).wait()
        pltpu.async_copy(rows_v.at[p], sp.at[ibuf.at[kslot, j2, 1]], ssem[p], add=True)

    def drain(kslot, j2, p):
        pltpu.make_async_copy(
            rows_v.at[p], sp.at[ibuf.at[kslot, j2, 1]], ssem[p]
        ).wait()

    _pipeline(e_hbm, base, NBLK_H, ibuf, isem, prep, fire, retire, drain)
    plsc.subcore_barrier()
    pltpu.sync_copy(
        sp.at[pl.ds(s * ZT, ZT)], out_hbm.at[c, pl.ds(s * ZT, ZT)]
    )


_segsum8 = pl.kernel(
    _segsum8_body,
    out_type=jax.ShapeDtypeStruct((NC, NPAD, 8), jnp.float32),
    mesh=_mesh(),
    compiler_params=_sc_params(),
    scratch_types=[
        pltpu.VMEM((2, IBLK, 2, LANE), jnp.int32),
        pltpu.VMEM((NSLOT, LANE, 8), jnp.float32),
        [pltpu.SemaphoreType.DMA] * 2,
        [pltpu.SemaphoreType.DMA] * NSLOT,
        [pltpu.SemaphoreType.DMA] * NSLOT,
        pltpu.VMEM_SHARED((NPAD, 8), jnp.float32),
    ],
)


def _segsum64_body(e_hbm, tab_hbm, zeros_hbm, out_hbm, ibuf, rows_v, isem, gsem, ssem, sp):
    c = lax.axis_index("c")
    s = lax.axis_index("s")

    pltpu.sync_copy(zeros_hbm.at[pl.ds(s * ZT, ZT)], sp.at[pl.ds(s * ZT, ZT)])
    plsc.subcore_barrier()

    base = s * T_E
    off = c * NPAD

    def prep(kslot, j2):
        for m in range(LANE // 16):
            v = ibuf[kslot, j2, 0, pl.ds(m * 16, 16)]
            ibuf[kslot, j2, 0, pl.ds(m * 16, 16)] = v + off

    def fire(kslot, j2, p):
        pltpu.async_copy(tab_hbm.at[ibuf.at[kslot, j2, 0]], rows_v.at[p], gsem[p])

    def retire(kslot, j2, p):
        pltpu.make_async_copy(
            tab_hbm.at[ibuf.at[kslot, j2, 0]], rows_v.at[p], gsem[p]
        ).wait()
        pltpu.async_copy(rows_v.at[p], sp.at[ibuf.at[kslot, j2, 1]], ssem[p], add=True)

    def drain(kslot, j2, p):
        pltpu.make_async_copy(
            rows_v.at[p], sp.at[ibuf.at[kslot, j2, 1]], ssem[p]
        ).wait()

    _pipeline(e_hbm, base, NBLK_E, ibuf, isem, prep, fire, retire, drain)
    plsc.subcore_barrier()
    pltpu.sync_copy(
        sp.at[pl.ds(s * ZT, ZT)], out_hbm.at[c, pl.ds(s * ZT, ZT)]
    )


_segsum64 = pl.kernel(
    _segsum64_body,
    out_type=jax.ShapeDtypeStruct((NC, NPAD, 32), jnp.float32),
    mesh=_mesh(),
    compiler_params=_sc_params(),
    scratch_types=[
        pltpu.VMEM((2, IBLK, 2, LANE), jnp.int32),
        pltpu.VMEM((NSLOT, LANE, 32), jnp.float32),
        [pltpu.SemaphoreType.DMA] * 2,
        [pltpu.SemaphoreType.DMA] * NSLOT,
        [pltpu.SemaphoreType.DMA] * NSLOT,
        pltpu.VMEM_SHARED((NPAD, 32), jnp.float32),
    ],
)


_RB = 6272


def _b_body(x_ref, dT_ref, o_ref):
    d = dT_ref[:, 0:1] + dT_ref[:, 1:2]
    no = lax.rsqrt(jnp.maximum(d, 1.0))
    xn = x_ref[...] * no
    zz = jnp.zeros((_RB, 1), jnp.float32)
    oo = jnp.ones((_RB, 1), jnp.float32)
    o_ref[...] = jnp.concatenate([xn, zz, oo], axis=1)


def _pass_b(xpad, degT):
    return pl.pallas_call(
        _b_body,
        grid=(NPAD // _RB,),
        in_specs=[
            pl.BlockSpec((_RB, 6), lambda i: (i, 0)),
            pl.BlockSpec((_RB, 2), lambda i: (i, 0)),
        ],
        out_specs=pl.BlockSpec((_RB, 8), lambda i: (i, 0)),
        out_shape=jax.ShapeDtypeStruct((NPAD, 8), jnp.float32),
    )(xpad, degT)


def _d_body(a0_ref, a1_ref, dT_ref, w_ref, b_ref, o_ref):
    a = a0_ref[0] + a1_ref[0]
    ni = lax.rsqrt(jnp.maximum(a[:, 7:8], 1.0))
    dout = dT_ref[:, 0:1] + dT_ref[:, 1:2]
    no = lax.rsqrt(jnp.maximum(dout, 1.0))
    h = jnp.dot(a, w_ref[...], preferred_element_type=jnp.float32)
    h = jnp.maximum(h * ni + b_ref[...], 0.0) * no
    o_ref[0] = h[:, 0:32]
    o_ref[1] = h[:, 32:64]


def _pass_d(agg1s, degT, W1p, b1r):
    return pl.pallas_call(
        _d_body,
        grid=(NPAD // _RB,),
        in_specs=[
            pl.BlockSpec((1, _RB, 8), lambda i: (0, i, 0)),
            pl.BlockSpec((1, _RB, 8), lambda i: (1, i, 0)),
            pl.BlockSpec((_RB, 2), lambda i: (i, 0)),
            pl.BlockSpec((8, 64), lambda i: (0, 0)),
            pl.BlockSpec((1, 64), lambda i: (0, 0)),
        ],
        out_specs=pl.BlockSpec((2, _RB, 32), lambda i: (0, i, 0)),
        out_shape=jax.ShapeDtypeStruct((2, NPAD, 32), jnp.float32),
    )(agg1s, agg1s, degT, W1p, b1r)


_RF = 2000


def _f_body(a2a_ref, a2b_ref, p0_ref, p1_ref, wa_ref, wb_ref, b_ref, o_ref):
    din = p0_ref[0][:, 7:8] + p1_ref[0][:, 7:8]
    ni = lax.rsqrt(jnp.maximum(din, 1.0))
    h = jnp.dot(a2a_ref[0] * ni, wa_ref[...], preferred_element_type=jnp.float32)
    h = h + jnp.dot(a2b_ref[0] * ni, wb_ref[...], preferred_element_type=jnp.float32)
    o_ref[...] = jnp.maximum(h + b_ref[...], 0.0)


def _pass_f(agg2s, agg1s, W2, b2r):
    return pl.pallas_call(
        _f_body,
        grid=(N_NODES // _RF,),
        in_specs=[
            pl.BlockSpec((1, _RF, 32), lambda i: (0, i, 0)),
            pl.BlockSpec((1, _RF, 32), lambda i: (1, i, 0)),
            pl.BlockSpec((1, _RF, 8), lambda i: (0, i, 0)),
            pl.BlockSpec((1, _RF, 8), lambda i: (1, i, 0)),
            pl.BlockSpec((32, 50), lambda i: (0, 0)),
            pl.BlockSpec((32, 50), lambda i: (1, 0)),
            pl.BlockSpec((1, 50), lambda i: (0, 0)),
        ],
        out_specs=pl.BlockSpec((_RF, 50), lambda i: (i, 0)),
        out_shape=jax.ShapeDtypeStruct((N_NODES, 50), jnp.float32),
    )(agg2s, agg2s, agg1s, agg1s, W2, W2, b2r)


def kernel(input_data, edge_index, W1, b1, W2, b2):
    npad_e = ROWS_P * LANE - N_EDGES
    fill = jnp.array([[PAD_SRC], [PAD_DST]], jnp.int32)
    epad = jnp.concatenate(
        [edge_index, jnp.broadcast_to(fill, (2, npad_e))], axis=1
    )
    e3 = epad.reshape(2, ROWS_P, LANE).transpose(1, 0, 2)
    deg2 = _deg_call(e3).reshape(NC, DEG_W)
    degT = jnp.transpose(deg2)[:NPAD]
    xpad = jnp.pad(input_data, ((0, NPAD - N_NODES), (0, 0)))
    xext = _pass_b(xpad, degT)
    z8 = jnp.zeros((NPAD, 8), jnp.float32)
    agg1s = _segsum8(e3, xext, z8)
    W1p = jnp.pad(W1, ((0, 2), (0, 0)))
    h1s = _pass_d(agg1s, degT, W1p, b1.reshape(1, 64))
    z32 = jnp.zeros((NPAD, 32), jnp.float32)
    agg2s = _segsum64(e3, h1s.reshape(2 * NPAD, 32), z32)
    return _pass_f(agg2s, agg1s, W2, b2.reshape(1, 50))


# speedup vs baseline: 14.0764x; 1.3040x over previous
import jax
import jax.numpy as jnp
from jax import lax
from jax.experimental import pallas as pl
from jax.experimental.pallas import tpu as pltpu
from jax.experimental.pallas import tpu_sc as plsc

N_NODES = 50000
N_EDGES = 800000
LANE = 128
NC, NS = 2, 16
ROWS_P = 6400
PAD_SRC = 50000
PAD_DST = 50100
NPAD = 50176
ZT = NPAD // NS
DEG_W = 51200
DEG_T = DEG_W // NS

IBLK_E = 20
IBLK_H = 10
T_E = ROWS_P // NS
NBLK_E = T_E // IBLK_E
T_H = ROWS_P // (NC * NS)
NBLK_H = T_H // IBLK_H


def _mesh():
    return plsc.VectorSubcoreMesh(
        core_axis_name="c", subcore_axis_name="s", num_cores=NC, num_subcores=NS
    )


def _sc_params():
    return pltpu.CompilerParams(use_tc_tiling_on_sc=False)


def _pipeline(e_hbm, base, nblk, ibuf, isem, prep, fire, retire, drain,
              nslot, rd, iblk):
    def iload(k_dyn, kslot):
        pltpu.async_copy(
            e_hbm.at[pl.ds(base + k_dyn * iblk, iblk)], ibuf.at[kslot], isem[kslot]
        )

    def iwait(k_dyn, kslot):
        pltpu.make_async_copy(
            e_hbm.at[pl.ds(base + k_dyn * iblk, iblk)], ibuf.at[kslot], isem[kslot]
        ).wait()

    iload(0, 0)

    def blk(g, _):
        for bb in range(2):
            k = g * 2 + bb
            iwait(k, bb)
            for j2 in range(iblk):
                p = j2 % nslot
                if j2 >= nslot:
                    drain(bb, j2 - nslot, p)
                else:
                    def _dr(bb=bb, j2=j2, p=p):
                        drain(1 - bb, iblk - nslot + j2, p)

                    pl.when(k >= 1)(_dr)
                prep(bb, j2)
                fire(bb, j2, p)
                q = (j2 - rd) % nslot
                if j2 >= rd:
                    retire(bb, j2 - rd, q)
                else:
                    def _rt(bb=bb, j2=j2, q=q):
                        retire(1 - bb, iblk - rd + j2, q)

                    pl.when(k >= 1)(_rt)
                if j2 == nslot - 1:
                    def _ld(k=k, bb=bb):
                        iload(k + 1, 1 - bb)

                    pl.when(k < nblk - 1)(_ld)
        return 0

    lax.fori_loop(0, nblk // 2, blk, 0)
    for dj in range(rd):
        j2 = iblk - rd + dj
        retire(1, j2, j2 % nslot)
    for dj in range(nslot):
        j2 = iblk - nslot + dj
        drain(1, j2, j2 % nslot)


def _deg_body(e_hbm, zeros_hbm, ones_hbm, out_hbm, ibuf, ones_r, isem, ssem, deg_sp):
    c = lax.axis_index("c")
    s = lax.axis_index("s")

    pltpu.sync_copy(zeros_hbm.at[pl.ds(s * ZT, ZT)], deg_sp.at[pl.ds(s * ZT, ZT)])
    pltpu.sync_copy(ones_hbm, ones_r)
    plsc.subcore_barrier()

    base = (c * NS + s) * T_H

    def prep(kslot, j2):
        pass

    def fire(kslot, j2, p):
        pltpu.async_copy(ones_r, deg_sp.at[ibuf.at[kslot, j2, 0]], ssem[p], add=True)

    def retire(kslot, j2, p):
        pass

    def drain(kslot, j2, p):
        pltpu.make_async_copy(ones_r, deg_sp.at[ibuf.at[kslot, j2, 0]], ssem[p]).wait()

    _pipeline(e_hbm, base, NBLK_H, ibuf, isem, prep, fire, retire, drain,
              nslot=10, rd=5, iblk=IBLK_H)
    plsc.subcore_barrier()
    pltpu.sync_copy(
        deg_sp.at[pl.ds(s * ZT, ZT)], out_hbm.at[c, pl.ds(s * ZT, ZT)]
    )


_deg_call = pl.kernel(
    _deg_body,
    out_type=jax.ShapeDtypeStruct((NC, NPAD, 8), jnp.float32),
    mesh=_mesh(),
    compiler_params=_sc_params(),
    scratch_types=[
        pltpu.VMEM((2, IBLK_H, 2, LANE), jnp.int32),
        pltpu.VMEM((LANE, 8), jnp.float32),
        [pltpu.SemaphoreType.DMA] * 2,
        [pltpu.SemaphoreType.DMA] * 10,
        pltpu.VMEM_SHARED((NPAD, 8), jnp.float32),
    ],
)


def _segsum8_body(e_hbm, tab_hbm, zeros_hbm, out_hbm, ibuf, rows_v, isem, gsem, ssem, sp):
    c = lax.axis_index("c")
    s = lax.axis_index("s")

    pltpu.sync_copy(zeros_hbm.at[pl.ds(s * ZT, ZT)], sp.at[pl.ds(s * ZT, ZT)])
    plsc.subcore_barrier()

    base = (c * NS + s) * T_H

    def prep(kslot, j2):
        pass

    def fire(kslot, j2, p):
        pltpu.async_copy(tab_hbm.at[ibuf.at[kslot, j2, 0]], rows_v.at[p], gsem[p])

    def retire(kslot, j2, p):
        pltpu.make_async_copy(
            tab_hbm.at[ibuf.at[kslot, j2, 0]], rows_v.at[p], gsem[p]
        ).wait()
        pltpu.async_copy(rows_v.at[p], sp.at[ibuf.at[kslot, j2, 1]], ssem[p], add=True)

    def drain(kslot, j2, p):
        pltpu.make_async_copy(
            rows_v.at[p], sp.at[ibuf.at[kslot, j2, 1]], ssem[p]
        ).wait()

    _pipeline(e_hbm, base, NBLK_H, ibuf, isem, prep, fire, retire, drain,
              nslot=10, rd=5, iblk=IBLK_H)
    plsc.subcore_barrier()
    pltpu.sync_copy(
        sp.at[pl.ds(s * ZT, ZT)], out_hbm.at[c, pl.ds(s * ZT, ZT)]
    )


_segsum8 = pl.kernel(
    _segsum8_body,
    out_type=jax.ShapeDtypeStruct((NC, NPAD, 8), jnp.float32),
    mesh=_mesh(),
    compiler_params=_sc_params(),
    scratch_types=[
        pltpu.VMEM((2, IBLK_H, 2, LANE), jnp.int32),
        pltpu.VMEM((10, LANE, 8), jnp.float32),
        [pltpu.SemaphoreType.DMA] * 2,
        [pltpu.SemaphoreType.DMA] * 10,
        [pltpu.SemaphoreType.DMA] * 10,
        pltpu.VMEM_SHARED((NPAD, 8), jnp.float32),
    ],
)


def _segsum64_body(e_hbm, tab_hbm, zeros_hbm, out_hbm, ibuf, rows_v, isem, gsem, ssem, sp):
    c = lax.axis_index("c")
    s = lax.axis_index("s")

    pltpu.sync_copy(zeros_hbm.at[pl.ds(s * ZT, ZT)], sp.at[pl.ds(s * ZT, ZT)])
    plsc.subcore_barrier()

    base = s * T_E

    def prep(kslot, j2):
        pass

    def fire(kslot, j2, p):
        pltpu.async_copy(tab_hbm.at[ibuf.at[kslot, j2, 0]], rows_v.at[p], gsem[p])

    def retire(kslot, j2, p):
        pltpu.make_async_copy(
            tab_hbm.at[ibuf.at[kslot, j2, 0]], rows_v.at[p], gsem[p]
        ).wait()
        pltpu.async_copy(rows_v.at[p], sp.at[ibuf.at[kslot, j2, 1]], ssem[p], add=True)

    def drain(kslot, j2, p):
        pltpu.make_async_copy(
            rows_v.at[p], sp.at[ibuf.at[kslot, j2, 1]], ssem[p]
        ).wait()

    _pipeline(e_hbm.at[c], base, NBLK_E, ibuf, isem, prep, fire, retire, drain,
              nslot=4, rd=1, iblk=IBLK_E)
    plsc.subcore_barrier()
    pltpu.sync_copy(
        sp.at[pl.ds(s * ZT, ZT)], out_hbm.at[c, pl.ds(s * ZT, ZT)]
    )


_segsum64 = pl.kernel(
    _segsum64_body,
    out_type=jax.ShapeDtypeStruct((NC, NPAD, 32), jnp.float32),
    mesh=_mesh(),
    compiler_params=_sc_params(),
    scratch_types=[
        pltpu.VMEM((2, IBLK_E, 2, LANE), jnp.int32),
        pltpu.VMEM((4, LANE, 32), jnp.float32),
        [pltpu.SemaphoreType.DMA] * 2,
        [pltpu.SemaphoreType.DMA] * 4,
        [pltpu.SemaphoreType.DMA] * 4,
        pltpu.VMEM_SHARED((NPAD, 32), jnp.float32),
    ],
)


_RB = 7168
_G = NPAD // _RB
_R8 = _RB * 8 // 128
_R32 = _RB * 32 // 128
_R64 = _RB * 64 // 128


def _b_body(x_ref, d0_ref, d1_ref, o_ref):
    no = lax.rsqrt(jnp.maximum(d0_ref[0] + d1_ref[0], 1.0))
    col = lax.broadcasted_iota(jnp.int32, (_R8, 128), 1) % 8
    x = x_ref[...]
    o_ref[...] = jnp.where(col == 7, 1.0, jnp.where(col == 6, no, x * no))


def _pass_b(xq, degq):
    return pl.pallas_call(
        _b_body,
        grid=(_G,),
        in_specs=[
            pl.BlockSpec((_R8, 128), lambda i: (i, 0)),
            pl.BlockSpec((1, _R8, 128), lambda i: (0, i, 0)),
            pl.BlockSpec((1, _R8, 128), lambda i: (1, i, 0)),
        ],
        out_specs=pl.BlockSpec((_R8, 128), lambda i: (i, 0)),
        out_shape=jax.ShapeDtypeStruct((NPAD * 8 // 128, 128), jnp.float32),
    )(xq, degq, degq)


def _d_body(a0_ref, a1_ref, xe_ref, wa_ref, wb_ref, b128_ref, c32_ref,
            ba_ref, bb_ref, o_ref):
    a = a0_ref[0] + a1_ref[0]
    degb = jnp.dot(a, b128_ref[...], preferred_element_type=jnp.float32)
    ni = lax.rsqrt(jnp.maximum(degb, 1.0))
    no32 = jnp.dot(xe_ref[...], c32_ref[...], preferred_element_type=jnp.float32)
    an = a * ni
    ha = jnp.dot(an, wa_ref[...], preferred_element_type=jnp.float32)
    hb = jnp.dot(an, wb_ref[...], preferred_element_type=jnp.float32)
    ha = jnp.maximum(ha + ba_ref[...], 0.0) * no32
    hb = jnp.maximum(hb + bb_ref[...], 0.0) * no32
    o_ref[0] = ha.reshape(_R8, 4, 128).reshape(_R32, 128)
    o_ref[1] = hb.reshape(_R8, 4, 128).reshape(_R32, 128)


def _pass_d(agg1q, xext, WbigA, WbigB, B128, C32, b1A, b1B):
    return pl.pallas_call(
        _d_body,
        grid=(_G,),
        in_specs=[
            pl.BlockSpec((1, _R8, 128), lambda i: (0, i, 0)),
            pl.BlockSpec((1, _R8, 128), lambda i: (1, i, 0)),
            pl.BlockSpec((_R8, 128), lambda i: (i, 0)),
            pl.BlockSpec((128, 512), lambda i: (0, 0)),
            pl.BlockSpec((128, 512), lambda i: (0, 0)),
            pl.BlockSpec((128, 128), lambda i: (0, 0)),
            pl.BlockSpec((128, 512), lambda i: (0, 0)),
            pl.BlockSpec((1, 512), lambda i: (0, 0)),
            pl.BlockSpec((1, 512), lambda i: (0, 0)),
        ],
        out_specs=pl.BlockSpec((2, _R32, 128), lambda i: (0, i, 0)),
        out_shape=jax.ShapeDtypeStruct((2, NPAD * 32 // 128, 128), jnp.float32),
    )(agg1q, agg1q, xext, WbigA, WbigB, B128, C32, b1A, b1B)


def _f_body(a2a_ref, a2b_ref, p0_ref, p1_ref, wa_ref, wb_ref, b50_ref, bb_ref, o_ref):
    a1 = p0_ref[0] + p1_ref[0]
    deg8 = jnp.dot(a1, b50_ref[...], preferred_element_type=jnp.float32)
    ni = lax.rsqrt(jnp.maximum(deg8, 1.0))
    xa = a2a_ref[0].reshape(_R8, 4, 128).reshape(_R8, 512)
    xb = a2b_ref[0].reshape(_R8, 4, 128).reshape(_R8, 512)
    y = jnp.dot(xa, wa_ref[...], preferred_element_type=jnp.float32)
    y = y + jnp.dot(xb, wb_ref[...], preferred_element_type=jnp.float32)
    o_ref[...] = jnp.maximum(ni * y + bb_ref[...], 0.0)


def _pass_f(agg2q, agg1q, W2A, W2B, B50, b2big):
    return pl.pallas_call(
        _f_body,
        grid=(_G,),
        in_specs=[
            pl.BlockSpec((1, _R32, 128), lambda i: (0, i, 0)),
            pl.BlockSpec((1, _R32, 128), lambda i: (1, i, 0)),
            pl.BlockSpec((1, _R8, 128), lambda i: (0, i, 0)),
            pl.BlockSpec((1, _R8, 128), lambda i: (1, i, 0)),
            pl.BlockSpec((512, 800), lambda i: (0, 0)),
            pl.BlockSpec((512, 800), lambda i: (0, 0)),
            pl.BlockSpec((128, 800), lambda i: (0, 0)),
            pl.BlockSpec((1, 800), lambda i: (0, 0)),
        ],
        out_specs=pl.BlockSpec((_R8, 800), lambda i: (i, 0)),
        out_shape=jax.ShapeDtypeStruct((NPAD * 8 // 128, 800), jnp.float32),
    )(agg2q, agg2q, agg1q, agg1q, W2A, W2B, B50, b2big)


def kernel(input_data, edge_index, W1, b1, W2, b2):
    npad_e = ROWS_P * LANE - N_EDGES
    fill = jnp.array([[PAD_SRC], [PAD_DST]], jnp.int32)
    epad = jnp.concatenate(
        [edge_index, jnp.broadcast_to(fill, (2, npad_e))], axis=1
    )
    e3 = epad.reshape(2, ROWS_P, LANE).transpose(1, 0, 2)
    off2 = jnp.array([NPAD, 0], jnp.int32).reshape(1, 2, 1)
    eb = jnp.stack([e3, e3 + off2], 0)
    z8 = jnp.zeros((NPAD, 8), jnp.float32)
    ones8 = jnp.ones((LANE, 8), jnp.float32)
    degs = _deg_call(e3, z8, ones8)
    degq = degs.reshape(2, NPAD * 8 // 128, 128)
    xq = jnp.pad(input_data, ((0, NPAD - N_NODES), (0, 2))).reshape(
        NPAD * 8 // 128, 128
    )
    xext = _pass_b(xq, degq)
    agg1s = _segsum8(e3, xext.reshape(NPAD, 8), z8)
    agg1q = agg1s.reshape(2, NPAD * 8 // 128, 128)

    eye16 = jnp.eye(16, dtype=jnp.float32)
    i128 = jnp.arange(128)
    W1p = jnp.pad(W1, ((0, 2), (0, 0)))
    WbigA = jnp.kron(eye16, W1p[:, :32])
    WbigB = jnp.kron(eye16, W1p[:, 32:])
    B128 = (i128[:, None] == (i128[None, :] // 8) * 8 + 7).astype(jnp.float32)
    C32 = (i128[:, None] == (jnp.arange(512)[None, :] // 32) * 8 + 6).astype(
        jnp.float32
    )
    b1A = jnp.tile(b1[:32], 16).reshape(1, 512)
    b1B = jnp.tile(b1[32:], 16).reshape(1, 512)
    h = _pass_d(agg1q, xext, WbigA, WbigB, B128, C32, b1A, b1B)

    z32 = jnp.zeros((NPAD, 32), jnp.float32)
    agg2s = _segsum64(eb, h.reshape(2 * NPAD, 32), z32)
    agg2q = agg2s.reshape(2, NPAD * 32 // 128, 128)

    W2A = jnp.kron(eye16, W2[:32])
    W2B = jnp.kron(eye16, W2[32:])
    B50 = (i128[:, None] == (jnp.arange(800)[None, :] // 50) * 8 + 7).astype(
        jnp.float32
    )
    b2big = jnp.tile(b2, 16).reshape(1, 800)
    y = _pass_f(agg2q, agg1q, W2A, W2B, B50, b2big)
    return y[: N_NODES // 16].reshape(N_NODES, 50)
